# baseline (speedup 1.0000x reference)
import functools

import jax
import jax.numpy as jnp
from jax import lax
from jax.experimental import pallas as pl
from jax.experimental.pallas import tpu as pltpu
from jax.experimental.pallas import tpu_sc as plsc

NC, NS, LANES = 2, 16, 16
NW = NC * NS
B, L, E = 16384, 100, 64
EC = E // LANES
ROWS_PER_W = B // NW
CHUNK = 16
NSTEPS = ROWS_PER_W // CHUNK
NBUF = 2
LHALF = L // 2


def _lane_sum(x):
    lanes = lax.iota(jnp.int32, LANES)
    dnums = lax.GatherDimensionNumbers(
        offset_dims=(), collapsed_slice_dims=(0,), start_index_map=(0,)
    )
    for sh in (1, 2, 4, 8):
        perm = lax.reshape(lanes ^ sh, (LANES, 1))
        x = x + lax.gather(x, perm, dnums, slice_sizes=(1,),
                           mode=lax.GatherScatterMode.PROMISE_IN_BOUNDS)
    return x


def _rsqrt(v):
    i = lax.bitcast_convert_type(v, jnp.int32)
    i = 0x5F3759DF - lax.shift_right_logical(i, 1)
    y = lax.bitcast_convert_type(i, jnp.float32)
    for _ in range(3):
        y = y * (1.5 - 0.5 * v * y * y)
    return y


_mesh = plsc.VectorSubcoreMesh(
    core_axis_name="c", subcore_axis_name="s", num_cores=NC, num_subcores=NS
)

_SCRATCH = [
    pltpu.VMEM((L, ROWS_PER_W), jnp.int32),
    pltpu.VMEM((NBUF, CHUNK, E), jnp.float32),
    pltpu.VMEM((CHUNK, E), jnp.float32),
    pltpu.VMEM((E,), jnp.float32),
    pltpu.VMEM((E,), jnp.float32),
    pltpu.SemaphoreType.DMA,
    pltpu.SemaphoreType.DMA,
    pltpu.SemaphoreType.DMA,
    pltpu.SemaphoreType.DMA,
    pltpu.SemaphoreType.DMA,
    pltpu.SemaphoreType.DMA,
]


def _embed_body(idx_hbm, table_hbm, gamma_hbm, beta_hbm, out_hbm,
                idx_v, acc_v, out_v, gamma_v, beta_v,
                isem, gsemA0, gsemB0, gsemA1, gsemB1, osem):
    wid = lax.axis_index("s") * NC + lax.axis_index("c")
    base = wid * ROWS_PER_W

    pltpu.sync_copy(gamma_hbm, gamma_v)
    pltpu.sync_copy(beta_hbm, beta_v)
    pltpu.async_copy(idx_hbm.at[:, pl.ds(base, ROWS_PER_W)], idx_v, isem).wait()

    zeros = jnp.zeros((LANES,), jnp.float32)

    def zero_acc(buf):
        def zbody(j, carry):
            for c in range(EC):
                acc_v[buf, j, pl.ds(c * LANES, LANES)] = zeros
            return carry
        lax.fori_loop(0, CHUNK, zbody, 0)

    def fire(t, buf, gsemA, gsemB):
        col0 = t * CHUNK

        def gbody(h, carry):
            pltpu.async_copy(
                table_hbm.at[idx_v.at[2 * h, pl.ds(col0, CHUNK)]],
                acc_v.at[buf], gsemA, add=True)
            pltpu.async_copy(
                table_hbm.at[idx_v.at[2 * h + 1, pl.ds(col0, CHUNK)]],
                acc_v.at[buf], gsemB, add=True)
            return carry
        lax.fori_loop(0, LHALF, gbody, 0)

    def drain(t, buf, gsemA, gsemB):
        col0 = t * CHUNK

        def wbody(h, carry):
            pltpu.make_async_copy(
                table_hbm.at[idx_v.at[2 * h, pl.ds(col0, CHUNK)]],
                acc_v.at[buf], gsemA).wait()
            pltpu.make_async_copy(
                table_hbm.at[idx_v.at[2 * h + 1, pl.ds(col0, CHUNK)]],
                acc_v.at[buf], gsemB).wait()
            return carry
        lax.fori_loop(0, LHALF, wbody, 0)

    def compute(t, buf):
        row0 = base + t * CHUNK

        def cbody(j, carry):
            m = [acc_v[buf, j, pl.ds(c * LANES, LANES)] * (1.0 / L)
                 for c in range(EC)]
            tot = m[0] + m[1] + m[2] + m[3]
            mu = _lane_sum(tot) * (1.0 / E)
            d = [m[c] - mu for c in range(EC)]
            ss = d[0] * d[0] + d[1] * d[1] + d[2] * d[2] + d[3] * d[3]
            var = _lane_sum(ss) * (1.0 / E) + 1e-5
            inv = _rsqrt(var)
            for c in range(EC):
                sl = pl.ds(c * LANES, LANES)
                out_v[j, sl] = d[c] * inv * gamma_v[sl] + beta_v[sl]
            return carry
        lax.fori_loop(0, CHUNK, cbody, 0)
        pltpu.async_copy(out_v, out_hbm.at[pl.ds(row0, CHUNK)], osem).wait()

    zero_acc(0)
    zero_acc(1)
    fire(0, 0, gsemA0, gsemB0)

    def step(t, carry):
        buf = lax.rem(t, NBUF)

        @pl.when(t + 1 < NSTEPS)
        def _():
            @pl.when(lax.rem(t + 1, NBUF) == 0)
            def _():
                fire(t + 1, 0, gsemA0, gsemB0)

            @pl.when(lax.rem(t + 1, NBUF) == 1)
            def _():
                fire(t + 1, 1, gsemA1, gsemB1)

        @pl.when(buf == 0)
        def _():
            drain(t, 0, gsemA0, gsemB0)

        @pl.when(buf == 1)
        def _():
            drain(t, 1, gsemA1, gsemB1)

        compute(t, buf)
        zero_acc(buf)
        return carry

    lax.fori_loop(0, NSTEPS, step, 0)


_embed_kernel = functools.partial(
    pl.kernel,
    out_type=jax.ShapeDtypeStruct((B, E), jnp.float32),
    mesh=_mesh,
    scratch_types=_SCRATCH,
    compiler_params=pltpu.CompilerParams(use_tc_tiling_on_sc=False),
)(_embed_body)


def kernel(input, table, gamma, beta):
    idx_t = input.astype(jnp.int32).T
    return _embed_kernel(idx_t, table, gamma, beta)


# speedup vs baseline: 1.0016x; 1.0016x over previous
import functools

import jax
import jax.numpy as jnp
from jax import lax
from jax.experimental import pallas as pl
from jax.experimental.pallas import tpu as pltpu
from jax.experimental.pallas import tpu_sc as plsc

NC, NS, LANES = 2, 16, 16
NW = NC * NS
B, L, E = 16384, 100, 64
EC = E // LANES
ROWS_PER_W = B // NW
CHUNK = 32
NSTEPS = ROWS_PER_W // CHUNK
NBUF = 2
LHALF = L // 2


def _lane_sum(x):
    lanes = lax.iota(jnp.int32, LANES)
    dnums = lax.GatherDimensionNumbers(
        offset_dims=(), collapsed_slice_dims=(0,), start_index_map=(0,)
    )
    for sh in (1, 2, 4, 8):
        perm = lax.reshape(lanes ^ sh, (LANES, 1))
        x = x + lax.gather(x, perm, dnums, slice_sizes=(1,),
                           mode=lax.GatherScatterMode.PROMISE_IN_BOUNDS)
    return x


def _rsqrt(v):
    i = lax.bitcast_convert_type(v, jnp.int32)
    i = 0x5F3759DF - lax.shift_right_logical(i, 1)
    y = lax.bitcast_convert_type(i, jnp.float32)
    for _ in range(3):
        y = y * (1.5 - 0.5 * v * y * y)
    return y


_mesh = plsc.VectorSubcoreMesh(
    core_axis_name="c", subcore_axis_name="s", num_cores=NC, num_subcores=NS
)

_SCRATCH = [
    pltpu.VMEM((L, ROWS_PER_W), jnp.int32),
    pltpu.VMEM((NBUF, CHUNK, E), jnp.float32),
    pltpu.VMEM((CHUNK, E), jnp.float32),
    pltpu.VMEM((E,), jnp.float32),
    pltpu.VMEM((E,), jnp.float32),
    pltpu.SemaphoreType.DMA,
    pltpu.SemaphoreType.DMA,
    pltpu.SemaphoreType.DMA,
    pltpu.SemaphoreType.DMA,
    pltpu.SemaphoreType.DMA,
    pltpu.SemaphoreType.DMA,
]


def _embed_body(idx_hbm, table_hbm, gamma_hbm, beta_hbm, out_hbm,
                idx_v, acc_v, out_v, gamma_v, beta_v,
                isem, gsemA0, gsemB0, gsemA1, gsemB1, osem):
    wid = lax.axis_index("s") * NC + lax.axis_index("c")
    base = wid * ROWS_PER_W

    pltpu.sync_copy(gamma_hbm, gamma_v)
    pltpu.sync_copy(beta_hbm, beta_v)
    pltpu.async_copy(idx_hbm.at[:, pl.ds(base, ROWS_PER_W)], idx_v, isem).wait()

    zeros = jnp.zeros((LANES,), jnp.float32)

    def zero_acc(buf):
        def zbody(j, carry):
            for c in range(EC):
                acc_v[buf, j, pl.ds(c * LANES, LANES)] = zeros
            return carry
        lax.fori_loop(0, CHUNK, zbody, 0)

    def fire(t, buf, gsemA, gsemB):
        col0 = t * CHUNK

        def gbody(h, carry):
            pltpu.async_copy(
                table_hbm.at[idx_v.at[2 * h, pl.ds(col0, CHUNK)]],
                acc_v.at[buf], gsemA, add=True)
            pltpu.async_copy(
                table_hbm.at[idx_v.at[2 * h + 1, pl.ds(col0, CHUNK)]],
                acc_v.at[buf], gsemB, add=True)
            return carry
        lax.fori_loop(0, LHALF, gbody, 0)

    def drain(t, buf, gsemA, gsemB):
        col0 = t * CHUNK

        def wbody(h, carry):
            pltpu.make_async_copy(
                table_hbm.at[idx_v.at[2 * h, pl.ds(col0, CHUNK)]],
                acc_v.at[buf], gsemA).wait()
            pltpu.make_async_copy(
                table_hbm.at[idx_v.at[2 * h + 1, pl.ds(col0, CHUNK)]],
                acc_v.at[buf], gsemB).wait()
            return carry
        lax.fori_loop(0, LHALF, wbody, 0)

    def compute(t, buf):
        row0 = base + t * CHUNK

        def cbody(j, carry):
            m = [acc_v[buf, j, pl.ds(c * LANES, LANES)] * (1.0 / L)
                 for c in range(EC)]
            tot = m[0] + m[1] + m[2] + m[3]
            mu = _lane_sum(tot) * (1.0 / E)
            d = [m[c] - mu for c in range(EC)]
            ss = d[0] * d[0] + d[1] * d[1] + d[2] * d[2] + d[3] * d[3]
            var = _lane_sum(ss) * (1.0 / E) + 1e-5
            inv = _rsqrt(var)
            for c in range(EC):
                sl = pl.ds(c * LANES, LANES)
                out_v[j, sl] = d[c] * inv * gamma_v[sl] + beta_v[sl]
            return carry
        lax.fori_loop(0, CHUNK, cbody, 0)
        pltpu.async_copy(out_v, out_hbm.at[pl.ds(row0, CHUNK)], osem).wait()

    zero_acc(0)
    zero_acc(1)
    fire(0, 0, gsemA0, gsemB0)

    def step(t, carry):
        buf = lax.rem(t, NBUF)

        @pl.when(t + 1 < NSTEPS)
        def _():
            @pl.when(lax.rem(t + 1, NBUF) == 0)
            def _():
                fire(t + 1, 0, gsemA0, gsemB0)

            @pl.when(lax.rem(t + 1, NBUF) == 1)
            def _():
                fire(t + 1, 1, gsemA1, gsemB1)

        @pl.when(buf == 0)
        def _():
            drain(t, 0, gsemA0, gsemB0)

        @pl.when(buf == 1)
        def _():
            drain(t, 1, gsemA1, gsemB1)

        compute(t, buf)
        zero_acc(buf)
        return carry

    lax.fori_loop(0, NSTEPS, step, 0)


_embed_kernel = functools.partial(
    pl.kernel,
    out_type=jax.ShapeDtypeStruct((B, E), jnp.float32),
    mesh=_mesh,
    scratch_types=_SCRATCH,
    compiler_params=pltpu.CompilerParams(use_tc_tiling_on_sc=False),
)(_embed_body)


def kernel(input, table, gamma, beta):
    idx_t = input.astype(jnp.int32).T
    return _embed_kernel(idx_t, table, gamma, beta)
